# baseline (speedup 1.0000x reference)
import functools

import jax
import jax.numpy as jnp
from jax import lax
from jax.experimental import pallas as pl
from jax.experimental.pallas import tpu as pltpu
from jax.experimental.pallas import tpu_sc as plsc

VOCAB = 100000
DIM = 128
B = 16384
C = 20
NNEG = 200
NIDX = C + NNEG
HALF = NIDX // 2
SLOT = 112
CPAD = 32
NPAD = 208
L = 16
KV = DIM // L


def _lane_take(v, idx):
    dnums = lax.GatherDimensionNumbers(
        offset_dims=(), collapsed_slice_dims=(0,), start_index_map=(0,))
    return lax.gather(v, idx[:, None], dnums, (1,),
                      mode=lax.GatherScatterMode.PROMISE_IN_BOUNDS)


def _sc_workers():
    try:
        info = plsc.get_sparse_core_info()
        return info.num_cores, info.num_subcores
    except Exception:
        return 2, 16


@functools.partial(jax.jit, static_argnames=())
def kernel(target_word, context_words, negative_words, in_emb, out_emb):
    NC, NS = _sc_workers()
    NW = NC * NS
    BPW = B // NW
    CB = 64
    NCH = BPW // CB

    def pack_bf16(t):
        w = lax.bitcast_convert_type(t, jnp.uint32)
        w = w + jnp.uint32(0x7FFF) + ((w >> 16) & jnp.uint32(1))
        packed = (w[:, 0::2] >> 16) | (w[:, 1::2] & jnp.uint32(0xFFFF0000))
        return lax.bitcast_convert_type(packed, jnp.int32)

    out_emb_w = pack_bf16(out_emb)
    in_emb_w = pack_bf16(in_emb)

    allidx = jnp.concatenate(
        [context_words.astype(jnp.int32), negative_words.astype(jnp.int32)],
        axis=1).reshape(B, 2, HALF)
    allidx = jnp.concatenate([allidx, allidx[:, :, HALF - 2:]], axis=2)
    allidx = allidx.reshape(B * 2 * SLOT)
    tgt_idx = target_word.astype(jnp.int32)

    mesh = plsc.VectorSubcoreMesh(core_axis_name="c", subcore_axis_name="s")

    @functools.partial(
        pl.kernel,
        mesh=mesh,
        compiler_params=pltpu.CompilerParams(
            needs_layout_passes=False, use_tc_tiling_on_sc=False),
        out_type=[
            jax.ShapeDtypeStruct((B, CPAD), jnp.float32),
            jax.ShapeDtypeStruct((B, NPAD), jnp.float32),
        ],
        scratch_types=[
            pltpu.VMEM((CB * 2 * SLOT,), jnp.int32),
            pltpu.VMEM((CB,), jnp.int32),
            pltpu.VMEM((CB, DIM // 2), jnp.int32),
            pltpu.VMEM((SLOT, DIM // 2), jnp.int32),
            pltpu.VMEM((SLOT, DIM // 2), jnp.int32),
            pltpu.VMEM((SLOT, DIM // 2), jnp.int32),
            pltpu.VMEM((SLOT, DIM // 2), jnp.int32),
            pltpu.VMEM((CB, CPAD), jnp.float32),
            pltpu.VMEM((CB, NPAD), jnp.float32),
            pltpu.SemaphoreType.DMA,
            pltpu.SemaphoreType.DMA,
            pltpu.SemaphoreType.DMA,
        ],
    )
    def run(allidx_h, tgtidx_h, inemb_h, outemb_h, ctx_o, neg_o,
            idx_v, ti_v, tg_v, ra0, ra1, rb0, rb1, sc_v, sn_v, sema, semb, tsem):
        wid = lax.axis_index("s") * NC + lax.axis_index("c")
        base_w = wid * BPW
        lanes = lax.iota(jnp.int32, L)
        lane0 = lanes == 0
        perms = [lanes ^ m for m in (8, 4, 2, 1)]

        def chunk_body(c, _):
            base = base_w + c * CB
            pltpu.sync_copy(
                allidx_h.at[pl.ds(base * 2 * SLOT, CB * 2 * SLOT)], idx_v)
            pltpu.sync_copy(tgtidx_h.at[pl.ds(base, CB)], ti_v)
            pltpu.async_copy(inemb_h.at[ti_v], tg_v, tsem).wait()

            def issue(i, r0, r1, sem):
                pltpu.async_copy(
                    outemb_h.at[idx_v.at[pl.ds(i * 2 * SLOT, SLOT)]],
                    r0, sem)
                pltpu.async_copy(
                    outemb_h.at[idx_v.at[pl.ds(i * 2 * SLOT + SLOT, SLOT)]],
                    r1, sem)

            def drain(i, r0, r1, sem):
                pltpu.make_async_copy(
                    outemb_h.at[idx_v.at[pl.ds(i * 2 * SLOT, SLOT)]],
                    r0, sem).wait()
                pltpu.make_async_copy(
                    outemb_h.at[idx_v.at[pl.ds(i * 2 * SLOT + SLOT, SLOT)]],
                    r1, sem).wait()

            def compute(i, r0v, r1v):
                tv = [plsc.bitcast(tg_v[i, pl.ds(k * L, L)],
                                   jnp.bfloat16) for k in range(KV // 2)]

                def dot_row(rv, r):
                    prods = [plsc.bitcast(rv[r, pl.ds(k * L, L)],
                                          jnp.bfloat16) * tv[k]
                             for k in range(KV // 2)]
                    while len(prods) > 1:
                        prods = [a + b for a, b in
                                 zip(prods[::2], prods[1::2])]
                    w = plsc.bitcast(prods[0], jnp.int32)
                    lo = plsc.bitcast(lax.shift_left(w, 16), jnp.float32)
                    hi = plsc.bitcast(w & jnp.int32(-65536), jnp.float32)
                    acc = lo + hi
                    for p in perms:
                        acc = acc + _lane_take(acc, p)
                    return acc

                def ctx_body(r):
                    sv = dot_row(r0v, r)
                    plsc.store_scatter(
                        sc_v,
                        [jnp.full((L,), i, jnp.int32),
                         jnp.full((L,), r, jnp.int32)],
                        sv, mask=lane0)

                def negA_body(r):
                    sv = dot_row(r0v, r)
                    plsc.store_scatter(
                        sn_v,
                        [jnp.full((L,), i, jnp.int32),
                         jnp.full((L,), r - C, jnp.int32)],
                        sv, mask=lane0)

                def negB_body(r):
                    sv = dot_row(r1v, r)
                    plsc.store_scatter(
                        sn_v,
                        [jnp.full((L,), i, jnp.int32),
                         jnp.full((L,), r + (HALF - C), jnp.int32)],
                        sv, mask=lane0)

                plsc.parallel_loop(0, C, unroll=4)(ctx_body)
                plsc.parallel_loop(C, HALF, unroll=6)(negA_body)
                plsc.parallel_loop(0, HALF, unroll=5)(negB_body)

            issue(0, ra0, ra1, sema)

            def pair_body(j, _):
                i0 = 2 * j
                issue(i0 + 1, rb0, rb1, semb)
                drain(i0, ra0, ra1, sema)
                compute(i0, ra0, ra1)

                @pl.when(j < CB // 2 - 1)
                def _():
                    issue(i0 + 2, ra0, ra1, sema)

                drain(i0 + 1, rb0, rb1, semb)
                compute(i0 + 1, rb0, rb1)
                return 0

            lax.fori_loop(0, CB // 2, pair_body, 0)
            pltpu.sync_copy(sc_v, ctx_o.at[pl.ds(base, CB)])
            pltpu.sync_copy(sn_v, neg_o.at[pl.ds(base, CB)])
            return 0

        lax.fori_loop(0, NCH, chunk_body, 0)

    ctx_pad, neg_pad = run(allidx, tgt_idx, in_emb_w, out_emb_w)
    return ctx_pad[:, :C], neg_pad[:, :NNEG]


# speedup vs baseline: 4.3926x; 4.3926x over previous
import functools

import jax
import jax.numpy as jnp
from jax import lax
from jax.experimental import pallas as pl
from jax.experimental.pallas import tpu as pltpu
from jax.experimental.pallas import tpu_sc as plsc

VOCAB = 100000
DIM = 128
B = 16384
C = 20
NNEG = 200
NIDX = C + NNEG
HALF = NIDX // 2
SLOT = 112
CPAD = 32
NPAD = 208
L = 16
KV = DIM // L


def _lane_take(v, idx):
    dnums = lax.GatherDimensionNumbers(
        offset_dims=(), collapsed_slice_dims=(0,), start_index_map=(0,))
    return lax.gather(v, idx[:, None], dnums, (1,),
                      mode=lax.GatherScatterMode.PROMISE_IN_BOUNDS)


def _sc_workers():
    try:
        info = plsc.get_sparse_core_info()
        return info.num_cores, info.num_subcores
    except Exception:
        return 2, 16


@functools.partial(jax.jit, static_argnames=())
def kernel(target_word, context_words, negative_words, in_emb, out_emb):
    NC, NS = _sc_workers()
    NW = NC * NS
    BPW = B // NW
    CB = 64
    NCH = BPW // CB

    def pack_bf16(t):
        w = lax.bitcast_convert_type(t, jnp.uint32)
        w = w + jnp.uint32(0x7FFF) + ((w >> 16) & jnp.uint32(1))
        packed = ((w[:, :DIM // 2] >> 16)
                  | (w[:, DIM // 2:] & jnp.uint32(0xFFFF0000)))
        return lax.bitcast_convert_type(packed, jnp.int32)

    out_emb_w = pack_bf16(out_emb)
    in_emb_w = pack_bf16(in_emb)

    allidx = jnp.concatenate(
        [context_words.astype(jnp.int32), negative_words.astype(jnp.int32)],
        axis=1).reshape(B, 2, HALF)
    allidx = jnp.concatenate([allidx, allidx[:, :, HALF - 2:]], axis=2)
    allidx = allidx.reshape(B * 2 * SLOT)
    tgt_idx = target_word.astype(jnp.int32)

    mesh = plsc.VectorSubcoreMesh(core_axis_name="c", subcore_axis_name="s")

    @functools.partial(
        pl.kernel,
        mesh=mesh,
        compiler_params=pltpu.CompilerParams(
            needs_layout_passes=False, use_tc_tiling_on_sc=False),
        out_type=[
            jax.ShapeDtypeStruct((B, CPAD), jnp.float32),
            jax.ShapeDtypeStruct((B, NPAD), jnp.float32),
        ],
        scratch_types=[
            pltpu.VMEM((CB * 2 * SLOT,), jnp.int32),
            pltpu.VMEM((CB,), jnp.int32),
            pltpu.VMEM((CB, DIM // 2), jnp.int32),
            pltpu.VMEM((SLOT, DIM // 2), jnp.int32),
            pltpu.VMEM((SLOT, DIM // 2), jnp.int32),
            pltpu.VMEM((SLOT, DIM // 2), jnp.int32),
            pltpu.VMEM((SLOT, DIM // 2), jnp.int32),
            pltpu.VMEM((CB, CPAD), jnp.float32),
            pltpu.VMEM((CB, NPAD), jnp.float32),
            pltpu.SemaphoreType.DMA,
            pltpu.SemaphoreType.DMA,
            pltpu.SemaphoreType.DMA,
        ],
    )
    def run(allidx_h, tgtidx_h, inemb_h, outemb_h, ctx_o, neg_o,
            idx_v, ti_v, tg_v, ra0, ra1, rb0, rb1, sc_v, sn_v, sema, semb, tsem):
        wid = lax.axis_index("s") * NC + lax.axis_index("c")
        base_w = wid * BPW
        lanes = lax.iota(jnp.int32, L)
        lane0 = lanes == 0
        perms = [lanes ^ m for m in (8, 4, 2, 1)]

        def chunk_body(c, _):
            base = base_w + c * CB
            pltpu.sync_copy(
                allidx_h.at[pl.ds(base * 2 * SLOT, CB * 2 * SLOT)], idx_v)
            pltpu.sync_copy(tgtidx_h.at[pl.ds(base, CB)], ti_v)
            pltpu.async_copy(inemb_h.at[ti_v], tg_v, tsem).wait()

            def issue(i, r0, r1, sem):
                pltpu.async_copy(
                    outemb_h.at[idx_v.at[pl.ds(i * 2 * SLOT, SLOT)]],
                    r0, sem)
                pltpu.async_copy(
                    outemb_h.at[idx_v.at[pl.ds(i * 2 * SLOT + SLOT, SLOT)]],
                    r1, sem)

            def drain(i, r0, r1, sem):
                pltpu.make_async_copy(
                    outemb_h.at[idx_v.at[pl.ds(i * 2 * SLOT, SLOT)]],
                    r0, sem).wait()
                pltpu.make_async_copy(
                    outemb_h.at[idx_v.at[pl.ds(i * 2 * SLOT + SLOT, SLOT)]],
                    r1, sem).wait()

            def compute(i, r0v, r1v):
                tv = [plsc.bitcast(tg_v[i, pl.ds(k * L, L)],
                                   jnp.bfloat16) for k in range(KV // 2)]

                def dot_row(rv, r):
                    prods = [plsc.bitcast(rv[r, pl.ds(k * L, L)],
                                          jnp.bfloat16) * tv[k]
                             for k in range(KV // 2)]
                    while len(prods) > 1:
                        prods = [a + b for a, b in
                                 zip(prods[::2], prods[1::2])]
                    w = plsc.bitcast(prods[0], jnp.int32)
                    lo = plsc.bitcast(lax.shift_left(w, 16), jnp.float32)
                    hi = plsc.bitcast(w & jnp.int32(-65536), jnp.float32)
                    acc = lo + hi
                    for p in perms:
                        acc = acc + _lane_take(acc, p)
                    return acc

                def ctx_body(r):
                    sv = dot_row(r0v, r)
                    plsc.store_scatter(
                        sc_v,
                        [jnp.full((L,), i, jnp.int32),
                         jnp.full((L,), r, jnp.int32)],
                        sv, mask=lane0)

                def negA_body(r):
                    sv = dot_row(r0v, r)
                    plsc.store_scatter(
                        sn_v,
                        [jnp.full((L,), i, jnp.int32),
                         jnp.full((L,), r - C, jnp.int32)],
                        sv, mask=lane0)

                def negB_body(r):
                    sv = dot_row(r1v, r)
                    plsc.store_scatter(
                        sn_v,
                        [jnp.full((L,), i, jnp.int32),
                         jnp.full((L,), r + (HALF - C), jnp.int32)],
                        sv, mask=lane0)

                plsc.parallel_loop(0, C, unroll=4)(ctx_body)
                plsc.parallel_loop(C, HALF, unroll=6)(negA_body)
                plsc.parallel_loop(0, HALF, unroll=5)(negB_body)

            issue(0, ra0, ra1, sema)

            def pair_body(j, _):
                i0 = 2 * j
                issue(i0 + 1, rb0, rb1, semb)
                drain(i0, ra0, ra1, sema)
                compute(i0, ra0, ra1)

                @pl.when(j < CB // 2 - 1)
                def _():
                    issue(i0 + 2, ra0, ra1, sema)

                drain(i0 + 1, rb0, rb1, semb)
                compute(i0 + 1, rb0, rb1)
                return 0

            lax.fori_loop(0, CB // 2, pair_body, 0)
            pltpu.sync_copy(sc_v, ctx_o.at[pl.ds(base, CB)])
            pltpu.sync_copy(sn_v, neg_o.at[pl.ds(base, CB)])
            return 0

        lax.fori_loop(0, NCH, chunk_body, 0)

    ctx_pad, neg_pad = run(allidx, tgt_idx, in_emb_w, out_emb_w)
    return ctx_pad[:, :C], neg_pad[:, :NNEG]


# speedup vs baseline: 4.9643x; 1.1302x over previous
import functools

import jax
import jax.numpy as jnp
from jax import lax
from jax.experimental import pallas as pl
from jax.experimental.pallas import tpu as pltpu
from jax.experimental.pallas import tpu_sc as plsc

VOCAB = 100000
DIM = 128
B = 16384
C = 20
NNEG = 200
NIDX = C + NNEG
HALF = NIDX // 2
SLOT = 112
CPAD = 32
NPAD = 208
L = 16
KV = DIM // L


def _lane_take(v, idx):
    dnums = lax.GatherDimensionNumbers(
        offset_dims=(), collapsed_slice_dims=(0,), start_index_map=(0,))
    return lax.gather(v, idx[:, None], dnums, (1,),
                      mode=lax.GatherScatterMode.PROMISE_IN_BOUNDS)


def _sc_workers():
    try:
        info = plsc.get_sparse_core_info()
        return info.num_cores, info.num_subcores
    except Exception:
        return 2, 16


@functools.partial(jax.jit, static_argnames=())
def kernel(target_word, context_words, negative_words, in_emb, out_emb):
    NC, NS = _sc_workers()
    NW = NC * NS
    BPW = B // NW
    CB = 64
    NCH = BPW // CB

    def pack_bf16(t):
        w = lax.bitcast_convert_type(t, jnp.uint32)
        w = w + jnp.uint32(0x7FFF) + ((w >> 16) & jnp.uint32(1))
        packed = ((w[:, :DIM // 2] >> 16)
                  | (w[:, DIM // 2:] & jnp.uint32(0xFFFF0000)))
        return lax.bitcast_convert_type(packed, jnp.int32)

    out_emb_w = pack_bf16(out_emb)

    allidx = jnp.concatenate(
        [context_words.astype(jnp.int32), negative_words.astype(jnp.int32)],
        axis=1).reshape(B, 2, HALF)
    allidx = jnp.concatenate([allidx, allidx[:, :, HALF - 2:]], axis=2)
    allidx = allidx.reshape(B * 2 * SLOT)
    tgt_idx = target_word.astype(jnp.int32)

    mesh = plsc.VectorSubcoreMesh(core_axis_name="c", subcore_axis_name="s")

    @functools.partial(
        pl.kernel,
        mesh=mesh,
        compiler_params=pltpu.CompilerParams(
            needs_layout_passes=False, use_tc_tiling_on_sc=False),
        out_type=[
            jax.ShapeDtypeStruct((B, CPAD), jnp.float32),
            jax.ShapeDtypeStruct((B, NPAD), jnp.float32),
        ],
        scratch_types=[
            pltpu.VMEM((CB * 2 * SLOT,), jnp.int32),
            pltpu.VMEM((CB,), jnp.int32),
            pltpu.VMEM((CB, DIM), jnp.float32),
            pltpu.VMEM((SLOT, DIM // 2), jnp.int32),
            pltpu.VMEM((SLOT, DIM // 2), jnp.int32),
            pltpu.VMEM((SLOT, DIM // 2), jnp.int32),
            pltpu.VMEM((SLOT, DIM // 2), jnp.int32),
            pltpu.VMEM((CB, CPAD), jnp.float32),
            pltpu.VMEM((CB, NPAD), jnp.float32),
            pltpu.SemaphoreType.DMA,
            pltpu.SemaphoreType.DMA,
            pltpu.SemaphoreType.DMA,
        ],
    )
    def run(allidx_h, tgtidx_h, inemb_h, outemb_h, ctx_o, neg_o,
            idx_v, ti_v, tg_v, ra0, ra1, rb0, rb1, sc_v, sn_v, sema, semb, tsem):
        wid = lax.axis_index("s") * NC + lax.axis_index("c")
        base_w = wid * BPW
        lanes = lax.iota(jnp.int32, L)
        lane0 = lanes == 0
        perms = [lanes ^ m for m in (8, 4, 2, 1)]

        def chunk_body(c, _):
            base = base_w + c * CB
            pltpu.sync_copy(
                allidx_h.at[pl.ds(base * 2 * SLOT, CB * 2 * SLOT)], idx_v)
            pltpu.sync_copy(tgtidx_h.at[pl.ds(base, CB)], ti_v)
            pltpu.async_copy(inemb_h.at[ti_v], tg_v, tsem).wait()

            def issue(i, r0, r1, sem):
                pltpu.async_copy(
                    outemb_h.at[idx_v.at[pl.ds(i * 2 * SLOT, SLOT)]],
                    r0, sem)
                pltpu.async_copy(
                    outemb_h.at[idx_v.at[pl.ds(i * 2 * SLOT + SLOT, SLOT)]],
                    r1, sem)

            def drain(i, r0, r1, sem):
                pltpu.make_async_copy(
                    outemb_h.at[idx_v.at[pl.ds(i * 2 * SLOT, SLOT)]],
                    r0, sem).wait()
                pltpu.make_async_copy(
                    outemb_h.at[idx_v.at[pl.ds(i * 2 * SLOT + SLOT, SLOT)]],
                    r1, sem).wait()

            def compute(i, r0v, r1v):
                tv = [plsc.pack(tg_v[i, pl.ds(k * L, L)],
                                tg_v[i, pl.ds(DIM // 2 + k * L, L)],
                                format=plsc.PackFormat.INTERLEAVED)
                      for k in range(KV // 2)]

                def dot_row(rv, r):
                    prods = [plsc.bitcast(rv[r, pl.ds(k * L, L)],
                                          jnp.bfloat16) * tv[k]
                             for k in range(KV // 2)]
                    while len(prods) > 1:
                        prods = [a + b for a, b in
                                 zip(prods[::2], prods[1::2])]
                    w = plsc.bitcast(prods[0], jnp.int32)
                    lo = plsc.bitcast(lax.shift_left(w, 16), jnp.float32)
                    hi = plsc.bitcast(w & jnp.int32(-65536), jnp.float32)
                    acc = lo + hi
                    for p in perms:
                        acc = acc + _lane_take(acc, p)
                    return acc

                def ctx_body(r):
                    sv = dot_row(r0v, r)
                    plsc.store_scatter(
                        sc_v,
                        [jnp.full((L,), i, jnp.int32),
                         jnp.full((L,), r, jnp.int32)],
                        sv, mask=lane0)

                def negA_body(r):
                    sv = dot_row(r0v, r)
                    plsc.store_scatter(
                        sn_v,
                        [jnp.full((L,), i, jnp.int32),
                         jnp.full((L,), r - C, jnp.int32)],
                        sv, mask=lane0)

                def negB_body(r):
                    sv = dot_row(r1v, r)
                    plsc.store_scatter(
                        sn_v,
                        [jnp.full((L,), i, jnp.int32),
                         jnp.full((L,), r + (HALF - C), jnp.int32)],
                        sv, mask=lane0)

                plsc.parallel_loop(0, C, unroll=4)(ctx_body)
                plsc.parallel_loop(C, HALF, unroll=6)(negA_body)
                plsc.parallel_loop(0, HALF, unroll=5)(negB_body)

            issue(0, ra0, ra1, sema)

            def pair_body(j, _):
                i0 = 2 * j
                issue(i0 + 1, rb0, rb1, semb)
                drain(i0, ra0, ra1, sema)
                compute(i0, ra0, ra1)

                @pl.when(j < CB // 2 - 1)
                def _():
                    issue(i0 + 2, ra0, ra1, sema)

                drain(i0 + 1, rb0, rb1, semb)
                compute(i0 + 1, rb0, rb1)
                return 0

            lax.fori_loop(0, CB // 2, pair_body, 0)
            pltpu.sync_copy(sc_v, ctx_o.at[pl.ds(base, CB)])
            pltpu.sync_copy(sn_v, neg_o.at[pl.ds(base, CB)])
            return 0

        lax.fori_loop(0, NCH, chunk_body, 0)

    ctx_pad, neg_pad = run(allidx, tgt_idx, in_emb, out_emb_w)
    return ctx_pad[:, :C], neg_pad[:, :NNEG]


# speedup vs baseline: 5.1538x; 1.0382x over previous
import functools

import jax
import jax.numpy as jnp
from jax import lax
from jax.experimental import pallas as pl
from jax.experimental.pallas import tpu as pltpu
from jax.experimental.pallas import tpu_sc as plsc

VOCAB = 100000
DIM = 128
B = 16384
C = 20
NNEG = 200
NIDX = C + NNEG
HALF = NIDX // 2
SLOT = 112
CPAD = 32
NPAD = 208
L = 16
KV = DIM // L


def _lane_take(v, idx):
    dnums = lax.GatherDimensionNumbers(
        offset_dims=(), collapsed_slice_dims=(0,), start_index_map=(0,))
    return lax.gather(v, idx[:, None], dnums, (1,),
                      mode=lax.GatherScatterMode.PROMISE_IN_BOUNDS)


def _sc_workers():
    try:
        info = plsc.get_sparse_core_info()
        return info.num_cores, info.num_subcores
    except Exception:
        return 2, 16


@functools.partial(jax.jit, static_argnames=())
def kernel(target_word, context_words, negative_words, in_emb, out_emb):
    NC, NS = _sc_workers()
    NW = NC * NS
    BPW = B // NW
    CB = 64
    NCH = BPW // CB

    def pack_bf16(t):
        w = lax.bitcast_convert_type(t, jnp.uint32)
        w = w + jnp.uint32(0x7FFF) + ((w >> 16) & jnp.uint32(1))
        packed = ((w[:, :DIM // 2] >> 16)
                  | (w[:, DIM // 2:] & jnp.uint32(0xFFFF0000)))
        return lax.bitcast_convert_type(packed, jnp.int32)

    out_emb_w = pack_bf16(out_emb)

    allidx = jnp.concatenate(
        [context_words.astype(jnp.int32), negative_words.astype(jnp.int32)],
        axis=1).reshape(B, 2, HALF)
    allidx = jnp.concatenate([allidx, allidx[:, :, HALF - 2:]], axis=2)
    allidx = allidx.reshape(B * 2 * SLOT)
    tgt_idx = target_word.astype(jnp.int32)

    mesh = plsc.VectorSubcoreMesh(core_axis_name="c", subcore_axis_name="s")

    @functools.partial(
        pl.kernel,
        mesh=mesh,
        compiler_params=pltpu.CompilerParams(
            needs_layout_passes=False, use_tc_tiling_on_sc=False),
        out_type=[
            jax.ShapeDtypeStruct((B * CPAD,), jnp.float32),
            jax.ShapeDtypeStruct((B * NPAD,), jnp.float32),
        ],
        scratch_types=[
            pltpu.VMEM((CB * 2 * SLOT,), jnp.int32),
            pltpu.VMEM((CB,), jnp.int32),
            pltpu.VMEM((CB, DIM), jnp.float32),
            pltpu.VMEM((SLOT, DIM // 2), jnp.int32),
            pltpu.VMEM((SLOT, DIM // 2), jnp.int32),
            pltpu.VMEM((SLOT, DIM // 2), jnp.int32),
            pltpu.VMEM((SLOT, DIM // 2), jnp.int32),
            pltpu.VMEM((CB * CPAD,), jnp.float32),
            pltpu.VMEM((CB * NPAD,), jnp.float32),
            pltpu.SemaphoreType.DMA,
            pltpu.SemaphoreType.DMA,
            pltpu.SemaphoreType.DMA,
        ],
    )
    def run(allidx_h, tgtidx_h, inemb_h, outemb_h, ctx_o, neg_o,
            idx_v, ti_v, tg_v, ra0, ra1, rb0, rb1, sc_v, sn_v, sema, semb, tsem):
        wid = lax.axis_index("s") * NC + lax.axis_index("c")
        base_w = wid * BPW
        lanes = lax.iota(jnp.int32, L)
        lane0 = lanes == 0
        perms = [lanes ^ m for m in (8, 4, 2, 1)]

        def chunk_body(c, _):
            base = base_w + c * CB
            pltpu.sync_copy(
                allidx_h.at[pl.ds(base * 2 * SLOT, CB * 2 * SLOT)], idx_v)
            pltpu.sync_copy(tgtidx_h.at[pl.ds(base, CB)], ti_v)
            pltpu.async_copy(inemb_h.at[ti_v], tg_v, tsem).wait()

            def issue(i, r0, r1, sem):
                pltpu.async_copy(
                    outemb_h.at[idx_v.at[pl.ds(i * 2 * SLOT, SLOT)]],
                    r0, sem)
                pltpu.async_copy(
                    outemb_h.at[idx_v.at[pl.ds(i * 2 * SLOT + SLOT, SLOT)]],
                    r1, sem)

            def drain(i, r0, r1, sem):
                pltpu.make_async_copy(
                    outemb_h.at[idx_v.at[pl.ds(i * 2 * SLOT, SLOT)]],
                    r0, sem).wait()
                pltpu.make_async_copy(
                    outemb_h.at[idx_v.at[pl.ds(i * 2 * SLOT + SLOT, SLOT)]],
                    r1, sem).wait()

            def compute(i, r0v, r1v):
                tv = [plsc.pack(tg_v[i, pl.ds(k * L, L)],
                                tg_v[i, pl.ds(DIM // 2 + k * L, L)],
                                format=plsc.PackFormat.INTERLEAVED)
                      for k in range(KV // 2)]

                def dot_row(rv, r):
                    prods = [plsc.bitcast(rv[r, pl.ds(k * L, L)],
                                          jnp.bfloat16) * tv[k]
                             for k in range(KV // 2)]
                    while len(prods) > 1:
                        prods = [a + b for a, b in
                                 zip(prods[::2], prods[1::2])]
                    w = plsc.bitcast(prods[0], jnp.int32)
                    lo = plsc.bitcast(lax.shift_left(w, 16), jnp.float32)
                    hi = plsc.bitcast(w & jnp.int32(-65536), jnp.float32)
                    acc = lo + hi
                    for p in perms:
                        acc = acc + _lane_take(acc, p)
                    return acc

                def ctx_body(r):
                    sv = dot_row(r0v, r)
                    plsc.store_scatter(
                        sc_v, [jnp.full((L,), i * CPAD + r, jnp.int32)],
                        sv, mask=lane0)

                def negA_body(r):
                    sv = dot_row(r0v, r)
                    plsc.store_scatter(
                        sn_v,
                        [jnp.full((L,), i * NPAD + (r - C), jnp.int32)],
                        sv, mask=lane0)

                def negB_body(r):
                    sv = dot_row(r1v, r)
                    plsc.store_scatter(
                        sn_v,
                        [jnp.full((L,), i * NPAD + (r + HALF - C),
                                  jnp.int32)],
                        sv, mask=lane0)

                plsc.parallel_loop(0, C, unroll=4)(ctx_body)
                plsc.parallel_loop(C, HALF, unroll=6)(negA_body)
                plsc.parallel_loop(0, HALF, unroll=5)(negB_body)

            issue(0, ra0, ra1, sema)

            def pair_body(j, _):
                i0 = 2 * j
                issue(i0 + 1, rb0, rb1, semb)
                drain(i0, ra0, ra1, sema)
                compute(i0, ra0, ra1)

                @pl.when(j < CB // 2 - 1)
                def _():
                    issue(i0 + 2, ra0, ra1, sema)

                drain(i0 + 1, rb0, rb1, semb)
                compute(i0 + 1, rb0, rb1)
                return 0

            lax.fori_loop(0, CB // 2, pair_body, 0)
            pltpu.sync_copy(sc_v, ctx_o.at[pl.ds(base * CPAD, CB * CPAD)])
            pltpu.sync_copy(sn_v, neg_o.at[pl.ds(base * NPAD, CB * NPAD)])
            return 0

        lax.fori_loop(0, NCH, chunk_body, 0)

    ctx_pad, neg_pad = run(allidx, tgt_idx, in_emb, out_emb_w)
    return (ctx_pad.reshape(B, CPAD)[:, :C],
            neg_pad.reshape(B, NPAD)[:, :NNEG])
